# baseline (speedup 1.0000x reference)
import jax
import jax.numpy as jnp
from jax import lax
from jax.experimental import pallas as pl
from jax.experimental.pallas import tpu as pltpu, tpu_sc as plsc

_MARGIN = 0.2
_EPS_INV = 1e12

_L = 16
_NC, _NS = 2, 16
_NW = _NC * _NS
_V, _D = 4096, 128
_T = 32768
_TPW = _T // _NW
_C = 64
_NCH = _TPW // _C
_RPW = _V // _NS


def _iota16():
    return lax.iota(jnp.int32, _L)


def _rsqrt16(x):
    i = plsc.bitcast(x, jnp.int32)
    i = jnp.int32(0x5F3759DF) - (i >> 1)
    y = plsc.bitcast(i, jnp.float32)
    for _ in range(3):
        y = y * (jnp.float32(1.5) - jnp.float32(0.5) * x * y * y)
    return jnp.minimum(y, jnp.float32(_EPS_INV))


def _colsum(ref, g):
    s = ref[pl.ds(g * _L, _L)]
    for l in range(1, _L):
        s = s + ref[pl.ds(l * _C + g * _L, _L)]
    return s


def _sc_body(emb_hbm, ai_hbm, pi_hbm, ni_hbm, out_hbm,
             buf, part_ap, part_an, invn, aidx, pidx, nidx,
             accv, shared_inv, sem0, sem1):
    cid = lax.axis_index("c")
    sid = lax.axis_index("s")
    wid = sid * _NC + cid
    cidx = _iota16() * _C

    row0 = sid * _RPW
    pltpu.async_copy(emb_hbm.at[pl.ds(row0, _C)], buf.at[0], sem0)
    for h in range(_RPW // _C):
        slot = h % 2
        pltpu.make_async_copy(emb_hbm.at[pl.ds(row0 + h * _C, _C)],
                              buf.at[slot], sem0).wait()
        if h < _RPW // _C - 1:
            pltpu.async_copy(emb_hbm.at[pl.ds(row0 + (h + 1) * _C, _C)],
                             buf.at[1 - slot], sem0)

        @plsc.parallel_loop(0, _C, unroll=2)
        def sq_row(r):
            s = [jnp.zeros((_L,), jnp.float32)] * 4
            for j in range(_D // _L):
                v = buf[slot, r, pl.ds(j * _L, _L)]
                s[j % 4] = s[j % 4] + v * v
            plsc.store_scatter(part_ap, [cidx + r], (s[0] + s[1]) + (s[2] + s[3]))

        for g in range(_C // _L):
            invn[pl.ds(row0 + h * _C + g * _L, _L)] = _rsqrt16(_colsum(part_ap, g))
    pltpu.sync_copy(invn.at[pl.ds(row0, _RPW)], shared_inv.at[pl.ds(row0, _RPW)])
    plsc.subcore_barrier()
    pltpu.sync_copy(shared_inv, invn)

    tbase = wid * _TPW
    pltpu.sync_copy(ai_hbm.at[pl.ds(tbase, _TPW)], aidx)
    pltpu.sync_copy(pi_hbm.at[pl.ds(tbase, _TPW)], pidx)
    pltpu.sync_copy(ni_hbm.at[pl.ds(tbase, _TPW)], nidx)

    def _gather(c, slot, sem):
        for k, idx in enumerate((aidx, pidx, nidx)):
            pltpu.async_copy(emb_hbm.at[idx.at[pl.ds(c * _C, _C)]],
                             buf.at[3 * slot + k], sem)

    def _gather_wait(c, slot, sem):
        for k, idx in enumerate((aidx, pidx, nidx)):
            pltpu.make_async_copy(emb_hbm.at[idx.at[pl.ds(c * _C, _C)]],
                                  buf.at[3 * slot + k], sem).wait()

    def _compute(c, slot, acc):
        a_ref = buf.at[3 * slot + 0]
        p_ref = buf.at[3 * slot + 1]
        n_ref = buf.at[3 * slot + 2]

        @plsc.parallel_loop(0, _C, unroll=2)
        def dot_row(r):
            ap0 = jnp.zeros((_L,), jnp.float32)
            ap1 = jnp.zeros((_L,), jnp.float32)
            an0 = jnp.zeros((_L,), jnp.float32)
            an1 = jnp.zeros((_L,), jnp.float32)
            for j in range(0, _D // _L, 2):
                va = a_ref[r, pl.ds(j * _L, _L)]
                vb = a_ref[r, pl.ds((j + 1) * _L, _L)]
                ap0 = ap0 + va * p_ref[r, pl.ds(j * _L, _L)]
                ap1 = ap1 + vb * p_ref[r, pl.ds((j + 1) * _L, _L)]
                an0 = an0 + va * n_ref[r, pl.ds(j * _L, _L)]
                an1 = an1 + vb * n_ref[r, pl.ds((j + 1) * _L, _L)]
            rcol = cidx + r
            plsc.store_scatter(part_ap, [rcol], ap0 + ap1)
            plsc.store_scatter(part_an, [rcol], an0 + an1)

        for g in range(_C // _L):
            sap = _colsum(part_ap, g)
            san = _colsum(part_an, g)
            src = pl.ds(c * _C + g * _L, _L)
            ia = plsc.load_gather(invn, [aidx[src]])
            ip = plsc.load_gather(invn, [pidx[src]])
            inn = plsc.load_gather(invn, [nidx[src]])
            d = jnp.float32(2.0) * ia * (san * inn - sap * ip)
            acc = acc + jnp.maximum(d + jnp.float32(_MARGIN), jnp.float32(0.0))
        return acc

    _gather(0, 0, sem0)
    _gather(1, 1, sem1)

    def chunk_pair(t, acc):
        c0 = 2 * t
        more = t < _NCH // 2 - 1

        _gather_wait(c0, 0, sem0)
        acc = _compute(c0, 0, acc)

        @pl.when(more)
        def _():
            _gather(c0 + 2, 0, sem0)

        _gather_wait(c0 + 1, 1, sem1)
        acc = _compute(c0 + 1, 1, acc)

        @pl.when(more)
        def _():
            _gather(c0 + 3, 1, sem1)

        return acc

    acc = lax.fori_loop(0, _NCH // 2, chunk_pair, jnp.zeros((_L,), jnp.float32))
    accv[:] = acc
    pltpu.sync_copy(accv, out_hbm.at[wid])


def kernel(embeddings, triplets):
    emb = embeddings.astype(jnp.float32)
    trip = triplets.astype(jnp.int32)
    ai, pi, ni = trip[:, 0], trip[:, 1], trip[:, 2]

    mesh = plsc.VectorSubcoreMesh(core_axis_name="c", subcore_axis_name="s",
                                  num_cores=_NC, num_subcores=_NS)
    sc = pl.kernel(
        _sc_body,
        out_type=jax.ShapeDtypeStruct((_NW, _L), jnp.float32),
        mesh=mesh,
        compiler_params=pltpu.CompilerParams(needs_layout_passes=False,
                                             use_tc_tiling_on_sc=True),
        scratch_types=[
            pltpu.VMEM((6, _C, _D), jnp.float32),
            pltpu.VMEM((_L * _C,), jnp.float32),
            pltpu.VMEM((_L * _C,), jnp.float32),
            pltpu.VMEM((_V,), jnp.float32),
            pltpu.VMEM((_TPW,), jnp.int32),
            pltpu.VMEM((_TPW,), jnp.int32),
            pltpu.VMEM((_TPW,), jnp.int32),
            pltpu.VMEM((_L,), jnp.float32),
            pltpu.VMEM_SHARED((_V,), jnp.float32),
            pltpu.SemaphoreType.DMA,
            pltpu.SemaphoreType.DMA,
        ],
    )
    partial = sc(emb, ai, pi, ni)
    loss = jnp.sum(partial) / jnp.float32(_T)
    return (loss, triplets.shape[0])


# speedup vs baseline: 1.1367x; 1.1367x over previous
import jax
import jax.numpy as jnp
from jax import lax
from jax.experimental import pallas as pl
from jax.experimental.pallas import tpu as pltpu, tpu_sc as plsc

_MARGIN = 0.2
_EPS_INV = 1e12

_L = 16
_NC, _NS = 2, 16
_NW = _NC * _NS
_V, _D = 4096, 128
_T = 32768
_TPW = _T // _NW
_C = 64
_NCH = _TPW // _C
_RPW = _V // _NS


def _iota16():
    return lax.iota(jnp.int32, _L)


def _rsqrt16(x):
    i = plsc.bitcast(x, jnp.int32)
    i = jnp.int32(0x5F3759DF) - (i >> 1)
    y = plsc.bitcast(i, jnp.float32)
    for _ in range(3):
        y = y * (jnp.float32(1.5) - jnp.float32(0.5) * x * y * y)
    return jnp.minimum(y, jnp.float32(_EPS_INV))


def _colsum(ref, g):
    rows = g * (_L * _L) + _iota16() * _L
    s = plsc.load_gather(ref, [rows])
    for l in range(1, _L):
        s = s + plsc.load_gather(ref, [rows + l])
    return s


def _sc_body(emb_hbm, ai_hbm, pi_hbm, ni_hbm, out_hbm,
             buf, part_ap, part_an, invn, aidx, pidx, nidx,
             accv, shared_inv, sem0, sem1):
    cid = lax.axis_index("c")
    sid = lax.axis_index("s")
    wid = sid * _NC + cid

    row0 = sid * _RPW
    pltpu.async_copy(emb_hbm.at[pl.ds(row0, _C)], buf.at[0], sem0)
    for h in range(_RPW // _C):
        slot = h % 2
        pltpu.make_async_copy(emb_hbm.at[pl.ds(row0 + h * _C, _C)],
                              buf.at[slot], sem0).wait()
        if h < _RPW // _C - 1:
            pltpu.async_copy(emb_hbm.at[pl.ds(row0 + (h + 1) * _C, _C)],
                             buf.at[1 - slot], sem0)

        @plsc.parallel_loop(0, _C, unroll=2)
        def sq_row(r):
            s = [jnp.zeros((_L,), jnp.float32)] * 4
            for j in range(_D // _L):
                v = buf[slot, r, pl.ds(j * _L, _L)]
                s[j % 4] = s[j % 4] + v * v
            part_ap[pl.ds(r * _L, _L)] = (s[0] + s[1]) + (s[2] + s[3])

        for g in range(_C // _L):
            invn[pl.ds(row0 + h * _C + g * _L, _L)] = _rsqrt16(_colsum(part_ap, g))
    pltpu.sync_copy(invn.at[pl.ds(row0, _RPW)], shared_inv.at[pl.ds(row0, _RPW)])
    plsc.subcore_barrier()
    pltpu.sync_copy(shared_inv, invn)

    tbase = wid * _TPW
    pltpu.sync_copy(ai_hbm.at[pl.ds(tbase, _TPW)], aidx)
    pltpu.sync_copy(pi_hbm.at[pl.ds(tbase, _TPW)], pidx)
    pltpu.sync_copy(ni_hbm.at[pl.ds(tbase, _TPW)], nidx)

    def _gather(c, slot, sem):
        for k, idx in enumerate((aidx, pidx, nidx)):
            pltpu.async_copy(emb_hbm.at[idx.at[pl.ds(c * _C, _C)]],
                             buf.at[3 * slot + k], sem)

    def _gather_wait(c, slot, sem):
        for k, idx in enumerate((aidx, pidx, nidx)):
            pltpu.make_async_copy(emb_hbm.at[idx.at[pl.ds(c * _C, _C)]],
                                  buf.at[3 * slot + k], sem).wait()

    def _compute(c, slot, acc):
        a_ref = buf.at[3 * slot + 0]
        p_ref = buf.at[3 * slot + 1]
        n_ref = buf.at[3 * slot + 2]

        @plsc.parallel_loop(0, _C, unroll=2)
        def dot_row(r):
            ap0 = jnp.zeros((_L,), jnp.float32)
            ap1 = jnp.zeros((_L,), jnp.float32)
            an0 = jnp.zeros((_L,), jnp.float32)
            an1 = jnp.zeros((_L,), jnp.float32)
            for j in range(0, _D // _L, 2):
                va = a_ref[r, pl.ds(j * _L, _L)]
                vb = a_ref[r, pl.ds((j + 1) * _L, _L)]
                ap0 = ap0 + va * p_ref[r, pl.ds(j * _L, _L)]
                ap1 = ap1 + vb * p_ref[r, pl.ds((j + 1) * _L, _L)]
                an0 = an0 + va * n_ref[r, pl.ds(j * _L, _L)]
                an1 = an1 + vb * n_ref[r, pl.ds((j + 1) * _L, _L)]
            part_ap[pl.ds(r * _L, _L)] = ap0 + ap1
            part_an[pl.ds(r * _L, _L)] = an0 + an1

        for g in range(_C // _L):
            sap = _colsum(part_ap, g)
            san = _colsum(part_an, g)
            src = pl.ds(c * _C + g * _L, _L)
            ia = plsc.load_gather(invn, [aidx[src]])
            ip = plsc.load_gather(invn, [pidx[src]])
            inn = plsc.load_gather(invn, [nidx[src]])
            d = jnp.float32(2.0) * ia * (san * inn - sap * ip)
            acc = acc + jnp.maximum(d + jnp.float32(_MARGIN), jnp.float32(0.0))
        return acc

    _gather(0, 0, sem0)
    _gather(1, 1, sem1)

    def chunk_pair(t, acc):
        c0 = 2 * t
        more = t < _NCH // 2 - 1

        _gather_wait(c0, 0, sem0)
        acc = _compute(c0, 0, acc)

        @pl.when(more)
        def _():
            _gather(c0 + 2, 0, sem0)

        _gather_wait(c0 + 1, 1, sem1)
        acc = _compute(c0 + 1, 1, acc)

        @pl.when(more)
        def _():
            _gather(c0 + 3, 1, sem1)

        return acc

    acc = lax.fori_loop(0, _NCH // 2, chunk_pair, jnp.zeros((_L,), jnp.float32))
    accv[:] = acc
    pltpu.sync_copy(accv, out_hbm.at[wid])


def kernel(embeddings, triplets):
    emb = embeddings.astype(jnp.float32)
    trip = triplets.astype(jnp.int32)
    ai, pi, ni = trip[:, 0], trip[:, 1], trip[:, 2]

    mesh = plsc.VectorSubcoreMesh(core_axis_name="c", subcore_axis_name="s",
                                  num_cores=_NC, num_subcores=_NS)
    sc = pl.kernel(
        _sc_body,
        out_type=jax.ShapeDtypeStruct((_NW, _L), jnp.float32),
        mesh=mesh,
        compiler_params=pltpu.CompilerParams(needs_layout_passes=False,
                                             use_tc_tiling_on_sc=True),
        scratch_types=[
            pltpu.VMEM((6, _C, _D), jnp.float32),
            pltpu.VMEM((_C * _L,), jnp.float32),
            pltpu.VMEM((_C * _L,), jnp.float32),
            pltpu.VMEM((_V,), jnp.float32),
            pltpu.VMEM((_TPW,), jnp.int32),
            pltpu.VMEM((_TPW,), jnp.int32),
            pltpu.VMEM((_TPW,), jnp.int32),
            pltpu.VMEM((_L,), jnp.float32),
            pltpu.VMEM_SHARED((_V,), jnp.float32),
            pltpu.SemaphoreType.DMA,
            pltpu.SemaphoreType.DMA,
        ],
    )
    partial = sc(emb, ai, pi, ni)
    loss = jnp.sum(partial) / jnp.float32(_T)
    return (loss, triplets.shape[0])


# speedup vs baseline: 1.2552x; 1.1042x over previous
import jax
import jax.numpy as jnp
from jax import lax
from jax.experimental import pallas as pl
from jax.experimental.pallas import tpu as pltpu, tpu_sc as plsc

_MARGIN = 0.2
_EPS_INV = 1e12

_L = 16
_NC, _NS = 2, 16
_NW = _NC * _NS
_V, _D = 4096, 128
_T = 32768
_TPW = _T // _NW
_C = 64
_NCH = _TPW // _C
_RPW = _V // _NS


def _iota16():
    return lax.iota(jnp.int32, _L)


def _rsqrt16(x):
    i = plsc.bitcast(x, jnp.int32)
    i = jnp.int32(0x5F3759DF) - (i >> 1)
    y = plsc.bitcast(i, jnp.float32)
    for _ in range(3):
        y = y * (jnp.float32(1.5) - jnp.float32(0.5) * x * y * y)
    return jnp.minimum(y, jnp.float32(_EPS_INV))


def _colsum(ref, g):
    rows = g * (_L * _L) + _iota16() * _L
    s = plsc.load_gather(ref, [rows])
    for l in range(1, _L):
        s = s + plsc.load_gather(ref, [rows + l])
    return s


def _sc_body(emb_hbm, ai_hbm, pi_hbm, ni_hbm, out_hbm,
             e16_hbm, buf, buf16, part_ap, part_an, invn, aidx, pidx, nidx,
             accv, b16, shared_inv, sem0, sem1):
    cid = lax.axis_index("c")
    sid = lax.axis_index("s")
    wid = sid * _NC + cid

    row0 = sid * _RPW
    pltpu.async_copy(emb_hbm.at[pl.ds(row0, _C)], buf.at[0], sem0)
    for h in range(_RPW // _C):
        slot = h % 2
        pltpu.make_async_copy(emb_hbm.at[pl.ds(row0 + h * _C, _C)],
                              buf.at[slot], sem0).wait()
        if h < _RPW // _C - 1:
            pltpu.async_copy(emb_hbm.at[pl.ds(row0 + (h + 1) * _C, _C)],
                             buf.at[1 - slot], sem0)

        @plsc.parallel_loop(0, _C, unroll=2)
        def sq_row(r):
            s = [jnp.zeros((_L,), jnp.float32)] * 4
            for w in range(_D // (2 * _L)):
                va = buf[slot, r, pl.ds(2 * w * _L, _L)]
                vb = buf[slot, r, pl.ds((2 * w + 1) * _L, _L)]
                s[(2 * w) % 4] = s[(2 * w) % 4] + va * va
                s[(2 * w + 1) % 4] = s[(2 * w + 1) % 4] + vb * vb
                pk = plsc.pack(va, vb, format=plsc.PackFormat.INTERLEAVED)
                b16[r, pl.ds(w * _L, _L)] = plsc.bitcast(pk, jnp.float32)
            part_ap[pl.ds(r * _L, _L)] = (s[0] + s[1]) + (s[2] + s[3])

        pltpu.sync_copy(b16, e16_hbm.at[pl.ds(row0 + h * _C, _C)])
        for g in range(_C // _L):
            invn[pl.ds(row0 + h * _C + g * _L, _L)] = _rsqrt16(_colsum(part_ap, g))
    pltpu.sync_copy(invn.at[pl.ds(row0, _RPW)], shared_inv.at[pl.ds(row0, _RPW)])
    plsc.subcore_barrier()
    pltpu.sync_copy(shared_inv, invn)

    tbase = wid * _TPW
    pltpu.sync_copy(ai_hbm.at[pl.ds(tbase, _TPW)], aidx)
    pltpu.sync_copy(pi_hbm.at[pl.ds(tbase, _TPW)], pidx)
    pltpu.sync_copy(ni_hbm.at[pl.ds(tbase, _TPW)], nidx)

    def _gather(c, slot, sem):
        for k, idx in enumerate((aidx, pidx, nidx)):
            pltpu.async_copy(e16_hbm.at[idx.at[pl.ds(c * _C, _C)]],
                             buf16.at[3 * slot + k], sem)

    def _gather_wait(c, slot, sem):
        for k, idx in enumerate((aidx, pidx, nidx)):
            pltpu.make_async_copy(e16_hbm.at[idx.at[pl.ds(c * _C, _C)]],
                                  buf16.at[3 * slot + k], sem).wait()

    def _unpack16(ref, r, w):
        v = ref[r, pl.ds(w * _L, _L)]
        return plsc.unpack(plsc.bitcast(v, jnp.bfloat16),
                           format=plsc.PackFormat.INTERLEAVED)

    def _compute(c, slot, acc):
        a_ref = buf16.at[3 * slot + 0]
        p_ref = buf16.at[3 * slot + 1]
        n_ref = buf16.at[3 * slot + 2]

        @plsc.parallel_loop(0, _C, unroll=2)
        def dot_row(r):
            ap0 = jnp.zeros((_L,), jnp.float32)
            ap1 = jnp.zeros((_L,), jnp.float32)
            an0 = jnp.zeros((_L,), jnp.float32)
            an1 = jnp.zeros((_L,), jnp.float32)
            for w in range(_D // (2 * _L)):
                va, vb = _unpack16(a_ref, r, w)
                vpa, vpb = _unpack16(p_ref, r, w)
                vna, vnb = _unpack16(n_ref, r, w)
                ap0 = ap0 + va * vpa
                ap1 = ap1 + vb * vpb
                an0 = an0 + va * vna
                an1 = an1 + vb * vnb
            part_ap[pl.ds(r * _L, _L)] = ap0 + ap1
            part_an[pl.ds(r * _L, _L)] = an0 + an1

        for g in range(_C // _L):
            sap = _colsum(part_ap, g)
            san = _colsum(part_an, g)
            src = pl.ds(c * _C + g * _L, _L)
            ia = plsc.load_gather(invn, [aidx[src]])
            ip = plsc.load_gather(invn, [pidx[src]])
            inn = plsc.load_gather(invn, [nidx[src]])
            d = jnp.float32(2.0) * ia * (san * inn - sap * ip)
            acc = acc + jnp.maximum(d + jnp.float32(_MARGIN), jnp.float32(0.0))
        return acc

    _gather(0, 0, sem0)
    _gather(1, 1, sem1)

    def chunk_pair(t, acc):
        c0 = 2 * t
        more = t < _NCH // 2 - 1

        _gather_wait(c0, 0, sem0)
        acc = _compute(c0, 0, acc)

        @pl.when(more)
        def _():
            _gather(c0 + 2, 0, sem0)

        _gather_wait(c0 + 1, 1, sem1)
        acc = _compute(c0 + 1, 1, acc)

        @pl.when(more)
        def _():
            _gather(c0 + 3, 1, sem1)

        return acc

    acc = lax.fori_loop(0, _NCH // 2, chunk_pair, jnp.zeros((_L,), jnp.float32))
    accv[:] = acc
    pltpu.sync_copy(accv, out_hbm.at[wid])


def kernel(embeddings, triplets):
    emb = embeddings.astype(jnp.float32)
    trip = triplets.astype(jnp.int32)
    ai, pi, ni = trip[:, 0], trip[:, 1], trip[:, 2]

    mesh = plsc.VectorSubcoreMesh(core_axis_name="c", subcore_axis_name="s",
                                  num_cores=_NC, num_subcores=_NS)
    sc = pl.kernel(
        _sc_body,
        out_type=jax.ShapeDtypeStruct((_NW, _L), jnp.float32),
        mesh=mesh,
        compiler_params=pltpu.CompilerParams(needs_layout_passes=False),
        scratch_types=[
            pltpu.HBM((_V, _D // 2), jnp.float32),
            pltpu.VMEM((2, _C, _D), jnp.float32),
            pltpu.VMEM((6, _C, _D // 2), jnp.float32),
            pltpu.VMEM((_C * _L,), jnp.float32),
            pltpu.VMEM((_C * _L,), jnp.float32),
            pltpu.VMEM((_V,), jnp.float32),
            pltpu.VMEM((_TPW,), jnp.int32),
            pltpu.VMEM((_TPW,), jnp.int32),
            pltpu.VMEM((_TPW,), jnp.int32),
            pltpu.VMEM((_L,), jnp.float32),
            pltpu.VMEM((_C, _D // 2), jnp.float32),
            pltpu.VMEM_SHARED((_V,), jnp.float32),
            pltpu.SemaphoreType.DMA,
            pltpu.SemaphoreType.DMA,
        ],
    )
    partial = sc(emb, ai, pi, ni)
    loss = jnp.sum(partial) / jnp.float32(_T)
    return (loss, triplets.shape[0])
